# baseline (speedup 1.0000x reference)
import functools
import jax
import jax.numpy as jnp
from jax import lax
from jax.experimental import pallas as pl
from jax.experimental.pallas import tpu as pltpu
from jax.experimental.pallas import tpu_sc as plsc

_N = 4096
_D = 64
_C = 8
_BM = 512
_NW = 32
_RW = _N // _NW


def _sc_gather_body(ft1p, ft2p, t1_hbm, t2_hbm, a_hbm, b_hbm,
                    t_v, idx_v, rows_v, sem):
    cid = lax.axis_index("c")
    sid = lax.axis_index("s")
    base = (sid * 2 + cid) * _RW
    lanes = lax.iota(jnp.int32, 16)
    for src, dst in ((t1_hbm, a_hbm), (t2_hbm, b_hbm)):
        pltpu.sync_copy(src.at[pl.ds(base, _RW)], t_v)
        for i in range(_RW // 16):
            t16 = t_v[pl.ds(16 * i, 16)]
            idx_v[pl.ds(16 * i, 16)] = (
                (t16 >> 1) + 4 * lanes + 4 * (base + 16 * i))
        table = ft1p if dst is a_hbm else ft2p
        pltpu.async_copy(table.at[idx_v], rows_v, sem).wait()
        pltpu.sync_copy(rows_v, dst.at[pl.ds(base, _RW)])


def _sc_gather(ft1p, ft2p, t1, t2):
    mesh = plsc.VectorSubcoreMesh(core_axis_name="c", subcore_axis_name="s")
    f = pl.kernel(
        _sc_gather_body,
        mesh=mesh,
        out_type=[
            jax.ShapeDtypeStruct((_N, 16), jnp.float32),
            jax.ShapeDtypeStruct((_N, 16), jnp.float32),
        ],
        scratch_types=[
            pltpu.VMEM((_RW,), jnp.int32),
            pltpu.VMEM((_RW,), jnp.int32),
            pltpu.VMEM((_RW, 16), jnp.float32),
            pltpu.SemaphoreType.DMA,
        ],
        compiler_params=pltpu.CompilerParams(use_tc_tiling_on_sc=False),
    )
    return f(ft1p, ft2p, t1, t2)


def _slot_kernel(f1_ref, f2_ref, a2_ref, b2_ref, t1_ref, t2c_ref, t2r_ref,
                 out_ref):
    f1 = f1_ref[...]
    f2 = f2_ref[...]
    a2 = a2_ref[...]
    b2 = b2_ref[...]
    t1 = t1_ref[...]
    t2c = t2c_ref[...]
    t2r = t2r_ref[...]

    half1 = lax.broadcasted_iota(jnp.int32, (_BM, 16), 1) // _C
    ma = jnp.where(half1 == (t1 & 1), a2, 0.0)
    half2 = lax.broadcasted_iota(jnp.int32, (_N, 16), 1) // _C
    mb = jnp.where(half2 == (t2c & 1), b2, 0.0)

    ka = lax.broadcasted_iota(jnp.int32, (16, 16), 0) % _C
    kb = lax.broadcasted_iota(jnp.int32, (16, 16), 1) % _C
    p = jnp.where(ka == kb, 1.0, 0.0).astype(jnp.bfloat16)

    c1 = jax.lax.dot_general(
        ma.astype(jnp.bfloat16), p, (((1,), (0,)), ((), ())),
        preferred_element_type=jnp.float32)
    cross = jax.lax.dot_general(
        c1.astype(jnp.bfloat16), mb.astype(jnp.bfloat16),
        (((1,), (1,)), ((), ())),
        preferred_element_type=jnp.float32)
    full = jax.lax.dot_general(
        f1, f2, (((1,), (1,)), ((), ())),
        preferred_element_type=jnp.float32)

    mask = t1 == t2r
    out_ref[...] = jnp.where(mask, full, cross)


@jax.jit
def kernel(ft_1, ft_2, type1, type2):
    t1 = type1.astype(jnp.int32)
    t2 = type2.astype(jnp.int32)
    a2, b2 = _sc_gather(ft_1.reshape(_N * 4, 16), ft_2.reshape(_N * 4, 16),
                        t1, t2)
    t1c = t1.reshape(_N, 1)
    t2c = t2.reshape(_N, 1)
    t2r = t2.reshape(1, _N)

    grid = (_N // _BM,)
    return pl.pallas_call(
        _slot_kernel,
        grid=grid,
        in_specs=[
            pl.BlockSpec((_BM, _D), lambda i: (i, 0)),
            pl.BlockSpec((_N, _D), lambda i: (0, 0)),
            pl.BlockSpec((_BM, 16), lambda i: (i, 0)),
            pl.BlockSpec((_N, 16), lambda i: (0, 0)),
            pl.BlockSpec((_BM, 1), lambda i: (i, 0)),
            pl.BlockSpec((_N, 1), lambda i: (0, 0)),
            pl.BlockSpec((1, _N), lambda i: (0, 0)),
        ],
        out_specs=pl.BlockSpec((_BM, _N), lambda i: (i, 0)),
        out_shape=jax.ShapeDtypeStruct((_N, _N), jnp.float32),
    )(ft_1, ft_2, a2, b2, t1c, t2c, t2r)


# speedup vs baseline: 1.4238x; 1.4238x over previous
import jax
import jax.numpy as jnp
from jax import lax
from jax.experimental import pallas as pl

_N = 4096
_D = 64
_C = 8
_BM = 512
_BN = 2048


def _slot_kernel(f1_ref, f2_ref, t1_ref, t2c_ref, t2r_ref, out_ref):
    f1 = f1_ref[...]
    f2 = f2_ref[...]
    t1 = t1_ref[...]
    t2c = t2c_ref[...]
    t2r = t2r_ref[...]

    slot1 = lax.broadcasted_iota(jnp.int32, (_BM, _D), 1) // _C
    m1 = jnp.where(slot1 == t1, f1, 0.0)
    slot2 = lax.broadcasted_iota(jnp.int32, (_BN, _D), 1) // _C
    m2 = jnp.where(slot2 == t2c, f2, 0.0)

    ka = lax.broadcasted_iota(jnp.int32, (_D, _D), 0) % _C
    kb = lax.broadcasted_iota(jnp.int32, (_D, _D), 1) % _C
    p = jnp.where(ka == kb, 1.0, 0.0).astype(jnp.bfloat16)

    c1 = jax.lax.dot_general(
        m1.astype(jnp.bfloat16), p, (((1,), (0,)), ((), ())),
        preferred_element_type=jnp.float32)
    cross = jax.lax.dot_general(
        c1.astype(jnp.bfloat16), m2.astype(jnp.bfloat16),
        (((1,), (1,)), ((), ())),
        preferred_element_type=jnp.float32)
    full = jax.lax.dot_general(
        f1, f2, (((1,), (1,)), ((), ())),
        preferred_element_type=jnp.float32)

    mask = t1 == t2r
    out_ref[...] = jnp.where(mask, full, cross)


@jax.jit
def kernel(ft_1, ft_2, type1, type2):
    t1c = type1.astype(jnp.int32).reshape(_N, 1)
    t2c = type2.astype(jnp.int32).reshape(_N, 1)
    t2r = type2.astype(jnp.int32).reshape(1, _N)

    grid = (_N // _BN, _N // _BM)
    return pl.pallas_call(
        _slot_kernel,
        grid=grid,
        in_specs=[
            pl.BlockSpec((_BM, _D), lambda j, i: (i, 0)),
            pl.BlockSpec((_BN, _D), lambda j, i: (j, 0)),
            pl.BlockSpec((_BM, 1), lambda j, i: (i, 0)),
            pl.BlockSpec((_BN, 1), lambda j, i: (j, 0)),
            pl.BlockSpec((1, _BN), lambda j, i: (0, j)),
        ],
        out_specs=pl.BlockSpec((_BM, _BN), lambda j, i: (i, j)),
        out_shape=jax.ShapeDtypeStruct((_N, _N), jnp.float32),
    )(ft_1, ft_2, t1c, t2c, t2r)


# speedup vs baseline: 1.7021x; 1.1955x over previous
import jax
import jax.numpy as jnp
from jax import lax
from jax.experimental import pallas as pl
from jax.experimental.pallas import tpu as pltpu

_N = 4096
_D = 64
_C = 8
_BM = 512
_GRID = _N // _BM
_SLOTS = 4


def _slot_kernel(f1_ref, f2_ref, t1_ref, t2c_ref, t2r_ref, out_hbm, buf, sem):
    i = pl.program_id(0)
    slot = lax.rem(i, _SLOTS)

    @pl.when(i >= _SLOTS)
    def _():
        pltpu.make_async_copy(
            buf.at[slot],
            out_hbm.at[pl.ds((i - _SLOTS) * _BM, _BM)],
            sem.at[slot],
        ).wait()

    f1 = f1_ref[...]
    f2 = f2_ref[...]
    t1 = t1_ref[...]
    t2c = t2c_ref[...]
    t2r = t2r_ref[...]

    slot1 = lax.broadcasted_iota(jnp.int32, (_BM, _D), 1) // _C
    m1 = jnp.where(slot1 == t1, f1, 0.0)
    slot2 = lax.broadcasted_iota(jnp.int32, (_N, _D), 1) // _C
    m2 = jnp.where(slot2 == t2c, f2, 0.0)

    ka = lax.broadcasted_iota(jnp.int32, (_D, _D), 0) % _C
    kb = lax.broadcasted_iota(jnp.int32, (_D, _D), 1) % _C
    p = jnp.where(ka == kb, 1.0, 0.0).astype(jnp.bfloat16)

    c1 = jax.lax.dot_general(
        m1.astype(jnp.bfloat16), p, (((1,), (0,)), ((), ())),
        preferred_element_type=jnp.float32)
    cross = jax.lax.dot_general(
        c1.astype(jnp.bfloat16), m2.astype(jnp.bfloat16),
        (((1,), (1,)), ((), ())),
        preferred_element_type=jnp.float32)
    full = jax.lax.dot_general(
        f1, f2, (((1,), (1,)), ((), ())),
        preferred_element_type=jnp.float32)

    mask = t1 == t2r
    buf[slot] = jnp.where(mask, full, cross)

    pltpu.make_async_copy(
        buf.at[slot], out_hbm.at[pl.ds(i * _BM, _BM)], sem.at[slot]
    ).start()

    @pl.when(i == _GRID - 1)
    def _():
        for d in range(_SLOTS):
            j = _GRID - _SLOTS + d
            s = j % _SLOTS
            pltpu.make_async_copy(
                buf.at[s], out_hbm.at[pl.ds(j * _BM, _BM)], sem.at[s]
            ).wait()


@jax.jit
def kernel(ft_1, ft_2, type1, type2):
    t1c = type1.astype(jnp.int32).reshape(_N, 1)
    t2c = type2.astype(jnp.int32).reshape(_N, 1)
    t2r = type2.astype(jnp.int32).reshape(1, _N)

    return pl.pallas_call(
        _slot_kernel,
        grid=(_GRID,),
        in_specs=[
            pl.BlockSpec((_BM, _D), lambda i: (i, 0)),
            pl.BlockSpec((_N, _D), lambda i: (0, 0)),
            pl.BlockSpec((_BM, 1), lambda i: (i, 0)),
            pl.BlockSpec((_N, 1), lambda i: (0, 0)),
            pl.BlockSpec((1, _N), lambda i: (0, 0)),
        ],
        out_specs=pl.BlockSpec(memory_space=pltpu.MemorySpace.HBM),
        out_shape=jax.ShapeDtypeStruct((_N, _N), jnp.float32),
        scratch_shapes=[
            pltpu.VMEM((_SLOTS, _BM, _N), jnp.float32),
            pltpu.SemaphoreType.DMA((_SLOTS,)),
        ],
        compiler_params=pltpu.CompilerParams(
            vmem_limit_bytes=100 * 1024 * 1024),
    )(ft_1, ft_2, t1c, t2c, t2r)
